# baseline (speedup 1.0000x reference)
import functools

import jax
import jax.numpy as jnp
from jax import lax
from jax.experimental import pallas as pl
from jax.experimental.pallas import tpu as pltpu
from jax.experimental.pallas import tpu_sc as plsc

B, K, V, H = 16384, 50, 100001, 256
CLAMP = 127.0
L = 16
NC, NS = 2, 16
NW = NC * NS
SPW = B // NW
BLK = 64
NBUF = 8
NBLK = SPW // BLK
HV = H // L
HW = H // (2 * L)
KP = 112
MASK_HI = -65536


def _body(idx_hbm, stm_hbm, emb_hbm, b1_hbm, wout_hbm, bout_hbm, out_hbm,
          idx_v, rows0, rows1, rows2, rows3, rows4, rows5, rows6, rows7,
          stm_v, b1_v, wout_v, bout_v, y_v,
          sem0, sem1, sem2, sem3, sem4, sem5, sem6, sem7):
    rows = (rows0, rows1, rows2, rows3, rows4, rows5, rows6, rows7)
    sems = (sem0, sem1, sem2, sem3, sem4, sem5, sem6, sem7)
    wid = lax.axis_index("s") * NC + lax.axis_index("c")
    base = wid * SPW

    pltpu.sync_copy(b1_hbm, b1_v)
    pltpu.sync_copy(wout_hbm, wout_v)
    pltpu.sync_copy(bout_hbm, bout_v)
    pltpu.sync_copy(stm_hbm.at[pl.ds(base, SPW)], stm_v)

    lane = lax.iota(jnp.int32, L)
    zero = jnp.zeros((L,), jnp.float32)
    bout_vec = bout_v[...]

    def issue(loc, buf, sem):
        pltpu.async_copy(emb_hbm.at[idx_v.at[loc]], buf, sem)

    def drain(buf, sem):
        pltpu.make_async_copy(emb_hbm.at[idx_v.at[0]], buf, sem).wait()

    def compute_sums(buf):
        b1c = tuple(b1_v[pl.ds(v * L, L)] for v in range(HV))

        def split(w):
            lo = plsc.bitcast(lax.shift_left(w, 16), jnp.float32)
            hi = plsc.bitcast(lax.bitwise_and(w, jnp.int32(MASK_HI)), jnp.float32)
            return lo, hi

        def rbody(r, carry):
            cw, cb = carry
            ncw = list(cw)
            ncb = list(cb)
            for g in range(HW):
                lo, hi = split(buf[r, pl.ds(g * L, L)])
                ncw[2 * g] = ncw[2 * g] + lo
                ncw[2 * g + 1] = ncw[2 * g + 1] + hi
                lo, hi = split(buf[r + K, pl.ds(g * L, L)])
                ncb[2 * g] = ncb[2 * g] + lo
                ncb[2 * g + 1] = ncb[2 * g + 1] + hi
            return (tuple(ncw), tuple(ncb))

        cw, cb = lax.fori_loop(0, K, rbody, (b1c, b1c), unroll=2)

        p_wa = p_wb = p_ba = p_bb = zero
        for v in range(HV):
            wa = wout_v[pl.ds(v * L, L)]
            wb = wout_v[pl.ds(H + v * L, L)]
            cwc = jnp.minimum(jnp.maximum(cw[v], 0.0), CLAMP)
            cbc = jnp.minimum(jnp.maximum(cb[v], 0.0), CLAMP)
            p_wa = p_wa + cwc * wa
            p_wb = p_wb + cwc * wb
            p_ba = p_ba + cbc * wa
            p_bb = p_bb + cbc * wb
        return jnp.sum(p_wa + p_bb), jnp.sum(p_ba + p_wb)

    def block(blk, _):
        pltpu.sync_copy(idx_hbm.at[pl.ds(base + blk * BLK, BLK), :], idx_v)
        for u in range(NBUF - 1):
            issue(u, rows[u], sems[u])

        def chunk(c, _2, blk=blk):
            def grp(q, carry, c=c):
                y1, y2 = carry
                for u in range(NBUF):
                    l = c * L + NBUF * q + u
                    nxt = l + NBUF - 1
                    nslot = (u + NBUF - 1) % NBUF

                    @pl.when(nxt < BLK)
                    def _(nxt=nxt, nslot=nslot):
                        issue(nxt, rows[nslot], sems[nslot])

                    drain(rows[u], sems[u])
                    s1, s2 = compute_sums(rows[u])
                    j = NBUF * q + u
                    y1 = jnp.where(lane == j, jnp.broadcast_to(s1, (L,)), y1)
                    y2 = jnp.where(lane == j, jnp.broadcast_to(s2, (L,)), y2)
                return (y1, y2)

            y1, y2 = lax.fori_loop(0, L // NBUF, grp, (zero, zero))
            off = blk * BLK + c * L
            stm_chunk = stm_v[pl.ds(off, L)]
            y_v[pl.ds(off, L)] = (
                jnp.where(stm_chunk == 0, y1, y2) + bout_vec)
            return 0

        lax.fori_loop(0, BLK // L, chunk, 0)
        return 0

    lax.fori_loop(0, NBLK, block, 0)

    pltpu.sync_copy(y_v, out_hbm.at[pl.ds(base, SPW)])


@jax.jit
def _run(idx, stm, emb_i32, b1p, woutp, bout16):
    mesh = plsc.VectorSubcoreMesh(core_axis_name="c", subcore_axis_name="s",
                                  num_cores=NC, num_subcores=NS)
    f = pl.kernel(
        _body,
        out_type=jax.ShapeDtypeStruct((B,), jnp.float32),
        mesh=mesh,
        compiler_params=pltpu.CompilerParams(needs_layout_passes=False),
        scratch_types=(
            [pltpu.VMEM((BLK, KP), jnp.int32)]
            + [pltpu.VMEM((KP, H // 2), jnp.int32) for _ in range(NBUF)]
            + [
                pltpu.VMEM((SPW,), jnp.int32),
                pltpu.VMEM((H,), jnp.float32),
                pltpu.VMEM((2 * H,), jnp.float32),
                pltpu.VMEM((L,), jnp.float32),
                pltpu.VMEM((SPW,), jnp.float32),
            ]
            + [pltpu.SemaphoreType.DMA for _ in range(NBUF)]
        ),
    )
    return f(idx, stm, emb_i32, b1p, woutp, bout16)


def _perm256(x):
    return x.reshape(H // (2 * L), L, 2).transpose(0, 2, 1).reshape(H)


def kernel(feats_w, feats_b, stm, emb, b1, W_out, b_out):
    idx = jnp.concatenate(
        [feats_w.astype(jnp.int32), feats_b.astype(jnp.int32),
         jnp.zeros((B, KP - 2 * K), jnp.int32)], axis=1)
    emb_i32 = lax.bitcast_convert_type(
        emb.astype(jnp.bfloat16).reshape(V, H // 2, 2), jnp.int32)
    b1p = _perm256(b1.astype(jnp.float32))
    w0 = W_out.reshape(2 * H).astype(jnp.float32)
    woutp = jnp.concatenate([_perm256(w0[:H]), _perm256(w0[H:])])
    bout16 = jnp.broadcast_to(b_out.astype(jnp.float32), (L,))
    return _run(idx, stm.astype(jnp.int32), emb_i32, b1p, woutp, bout16)


# speedup vs baseline: 3.9406x; 3.9406x over previous
import functools

import jax
import jax.numpy as jnp
from jax import lax
from jax.experimental import pallas as pl
from jax.experimental.pallas import tpu as pltpu
from jax.experimental.pallas import tpu_sc as plsc

B, K, V, H = 16384, 50, 100001, 256
CLAMP = 127.0
L = 16
NC, NS = 2, 16
NW = NC * NS
VP = 100008
NPASS = H // (2 * NW)
CL = 128
KP = 56
NCH = B // CL
FLUSH_CH = 16
FB = FLUSH_CH * CL
MASK_HI = -65536
BT = 512


def _sc_body(idx2_hbm, embT_hbm, out_hbm,
             col_v, idxb0, idxb1, olo, ohi, semc, semi0, semi1):
    idxb = (idxb0, idxb1)
    semi = (semi0, semi1)
    wid = lax.axis_index("s") * NC + lax.axis_index("c")

    zero = jnp.zeros((L,), jnp.float32)

    def split(w):
        lo = plsc.bitcast(lax.shift_left(w, 16), jnp.float32)
        hi = plsc.bitcast(lax.bitwise_and(w, jnp.int32(MASK_HI)), jnp.float32)
        return lo, hi

    def issue_idx(ch, side_off, buf, sem):
        base = (ch * 2 * KP + side_off) * CL
        pltpu.async_copy(idx2_hbm.at[pl.ds(base, KP * CL)], buf, sem)

    def drain_idx(buf, sem):
        pltpu.make_async_copy(
            idx2_hbm.at[pl.ds(0, KP * CL)], buf, sem).wait()

    def chunk(buf, ch):
        for sub in range(CL // L):
            def kbody(k, carry):
                alo, ahi = carry
                vals = plsc.load_gather(
                    col_v, [buf[pl.ds(k * CL + sub * L, L)]])
                lo, hi = split(vals)
                return (alo + lo, ahi + hi)

            alo, ahi = lax.fori_loop(0, K, kbody, (zero, zero), unroll=5)
            s_loc = lax.rem(ch, FLUSH_CH) * CL + sub * L
            olo[pl.ds(s_loc, L)] = alo
            ohi[pl.ds(s_loc, L)] = ahi

    def one_pass(p2, _):
        r = (p2 // 2) * NW + wid
        side_off = lax.rem(p2, 2) * KP
        pltpu.async_copy(embT_hbm.at[pl.ds(r * VP, VP)], col_v, semc).wait()
        issue_idx(0, side_off, idxb[0], semi[0])

        def cpair(h, _2):
            ch0 = 2 * h
            issue_idx(ch0 + 1, side_off, idxb[1], semi[1])
            drain_idx(idxb[0], semi[0])
            chunk(idxb[0], ch0)

            @pl.when(ch0 + 2 < NCH)
            def _():
                issue_idx(ch0 + 2, side_off, idxb[0], semi[0])

            drain_idx(idxb[1], semi[1])
            chunk(idxb[1], ch0 + 1)

            @pl.when(lax.rem(h, FLUSH_CH // 2) == FLUSH_CH // 2 - 1)
            def _():
                soff = (h // (FLUSH_CH // 2)) * FB
                side = lax.rem(p2, 2)
                lobase = side * H * B + 2 * r * B + soff
                hibase = side * H * B + (2 * r + 1) * B + soff
                pltpu.sync_copy(olo, out_hbm.at[pl.ds(lobase, FB)])
                pltpu.sync_copy(ohi, out_hbm.at[pl.ds(hibase, FB)])
            return 0

        lax.fori_loop(0, NCH // 2, cpair, 0)
        return 0

    lax.fori_loop(0, 2 * NPASS, one_pass, 0)


def _tail_body(accw_ref, accb_ref, stm_ref, b1_ref, wa_ref, wb_ref, bout_ref,
               y_ref):
    b1c = b1_ref[0, 0, :][:, None]
    xw = jnp.clip(accw_ref[...] + b1c, 0.0, CLAMP)
    xb = jnp.clip(accb_ref[...] + b1c, 0.0, CLAMP)
    wa = wa_ref[0, 0, :][:, None]
    wb = wb_ref[0, 0, :][:, None]
    s1 = jnp.sum(xw * wa + xb * wb, axis=0)
    s2 = jnp.sum(xw * wb + xb * wa, axis=0)
    stm = stm_ref[0, 0, :]
    y_ref[0, 0, :] = jnp.where(stm == 0, s1, s2) + bout_ref[0, 0, 0]


@jax.jit
def _run(idx2, embT, stm2, b1c, wa, wb, bout):
    mesh = plsc.VectorSubcoreMesh(core_axis_name="c", subcore_axis_name="s",
                                  num_cores=NC, num_subcores=NS)
    sc = pl.kernel(
        _sc_body,
        out_type=jax.ShapeDtypeStruct((2 * H * B,), jnp.float32),
        mesh=mesh,
        compiler_params=pltpu.CompilerParams(needs_layout_passes=False),
        scratch_types=[
            pltpu.VMEM((VP,), jnp.int32),
            pltpu.VMEM((KP * CL,), jnp.int32),
            pltpu.VMEM((KP * CL,), jnp.int32),
            pltpu.VMEM((FB,), jnp.float32),
            pltpu.VMEM((FB,), jnp.float32),
            pltpu.SemaphoreType.DMA,
            pltpu.SemaphoreType.DMA,
            pltpu.SemaphoreType.DMA,
        ],
    )
    outwb = sc(idx2, embT)
    acc = outwb.reshape(2, H, B)
    accw = acc[0]
    accb = acc[1]

    tail = pl.pallas_call(
        _tail_body,
        out_shape=jax.ShapeDtypeStruct((B // BT, 1, BT), jnp.float32),
        grid=(B // BT,),
        in_specs=[
            pl.BlockSpec((H, BT), lambda i: (0, i)),
            pl.BlockSpec((H, BT), lambda i: (0, i)),
            pl.BlockSpec((1, 1, BT), lambda i: (i, 0, 0)),
            pl.BlockSpec((1, 1, H), lambda i: (0, 0, 0)),
            pl.BlockSpec((1, 1, H), lambda i: (0, 0, 0)),
            pl.BlockSpec((1, 1, H), lambda i: (0, 0, 0)),
            pl.BlockSpec((1, 1, 8), lambda i: (0, 0, 0)),
        ],
        out_specs=pl.BlockSpec((1, 1, BT), lambda i: (i, 0, 0)),
    )
    y = tail(accw, accb, stm2, b1c, wa, wb, bout)
    return y.reshape(B)


def kernel(feats_w, feats_b, stm, emb, b1, W_out, b_out):
    def _side(f):
        f3 = f.astype(jnp.int32).reshape(NCH, CL, K).transpose(0, 2, 1)
        return jnp.concatenate(
            [f3, jnp.zeros((NCH, KP - K, CL), jnp.int32)], axis=1)
    idx2 = jnp.concatenate([_side(feats_w), _side(feats_b)],
                           axis=1).reshape(-1)
    emb_i32 = lax.bitcast_convert_type(
        emb.astype(jnp.bfloat16).reshape(V, H // 2, 2), jnp.int32)
    embT = jnp.concatenate(
        [emb_i32, jnp.zeros((VP - V, H // 2), jnp.int32)],
        axis=0).T.reshape(-1)
    w0 = W_out.reshape(2 * H).astype(jnp.float32)
    return _run(idx2, embT, stm.astype(jnp.int32).reshape(B // BT, 1, BT),
                b1.astype(jnp.float32).reshape(1, 1, H),
                w0[:H].reshape(1, 1, H), w0[H:].reshape(1, 1, H),
                jnp.broadcast_to(b_out.astype(jnp.float32),
                                 (8,)).reshape(1, 1, 8))
